# baseline (speedup 1.0000x reference)
import jax
import jax.numpy as jnp
from jax import lax
from jax.experimental import pallas as pl
from jax.experimental.pallas import tpu as pltpu
from jax.experimental.pallas import tpu_sc as plsc

_B = 16384
_D = 32
_V = 100001
_NC = 2
_NS = 16
_NW = _NC * _NS
_BB = 2048
_H = _B // 2


def _make_gather(nb, off):
    bpw = nb // _NW

    def body(idx_ref, tab_ref, xout_ref, idx_v, fidx_v, rows_v, sem):
        wid = lax.axis_index("s") * _NC + lax.axis_index("c")
        base = wid * bpw
        pltpu.sync_copy(idx_ref.at[pl.ds(off + base, bpw)], idx_v)
        for k in range(bpw // 16):
            sl = pl.ds(16 * k, 16)
            v = idx_v[sl]
            for d in range(_D):
                fidx_v[d, sl] = v + jnp.int32(d * _V)
        copies = []
        for d in range(_D):
            copies.append(pltpu.async_copy(
                tab_ref.at[fidx_v.at[d]], rows_v.at[d], sem))
        for c in copies:
            c.wait()
        pltpu.sync_copy(rows_v, xout_ref.at[:, pl.ds(base, bpw)])

    return pl.kernel(
        body,
        out_type=jax.ShapeDtypeStruct((_D, nb), jnp.float32),
        mesh=plsc.VectorSubcoreMesh(core_axis_name="c", subcore_axis_name="s"),
        scratch_types=[
            pltpu.VMEM((bpw,), jnp.int32),
            pltpu.VMEM((_D, bpw), jnp.int32),
            pltpu.VMEM((_D, bpw), jnp.float32),
            pltpu.SemaphoreType.DMA,
        ],
        compiler_params=pltpu.CompilerParams(use_tc_tiling_on_sc=False),
    )


_gather_u = _make_gather(_B, 0)
_gather_m0 = _make_gather(_H, 0)
_gather_m1 = _make_gather(_H, _H)


def _mlp_body(xu_ref, xm_ref, w1_ref, b1_ref, w2_ref, b2_ref,
              w3_ref, b3_ref, out_ref):
    cdims = (((0,), (0,)), ((), ()))
    h1 = jnp.maximum(
        lax.dot_general(w1_ref[:_D], xu_ref[...], cdims,
                        preferred_element_type=jnp.float32)
        + lax.dot_general(w1_ref[_D:], xm_ref[...], cdims,
                          preferred_element_type=jnp.float32)
        + b1_ref[...][:, None], 0.0)
    h2 = jnp.maximum(
        lax.dot_general(w2_ref[...], h1, cdims,
                        preferred_element_type=jnp.float32)
        + b2_ref[...][:, None], 0.0)
    out_ref[...] = (
        lax.dot_general(w3_ref[...], h2, cdims,
                        preferred_element_type=jnp.float32)
        + b3_ref[...][:, None])


def _make_mlp(uoff):
    return pl.pallas_call(
        _mlp_body,
        grid=(_H // _BB,),
        in_specs=[
            pl.BlockSpec((_D, _BB), lambda i: (0, i + uoff)),
            pl.BlockSpec((_D, _BB), lambda i: (0, i)),
            pl.BlockSpec((2 * _D, 256), lambda i: (0, 0)),
            pl.BlockSpec((256,), lambda i: (0,)),
            pl.BlockSpec((256, 64), lambda i: (0, 0)),
            pl.BlockSpec((64,), lambda i: (0,)),
            pl.BlockSpec((64, 1), lambda i: (0, 0)),
            pl.BlockSpec((1,), lambda i: (0,)),
        ],
        out_specs=pl.BlockSpec((1, _BB), lambda i: (0, i)),
        out_shape=jax.ShapeDtypeStruct((1, _H), jnp.float32),
    )


_mlp_lo = _make_mlp(0)
_mlp_hi = _make_mlp(_H // _BB)


@jax.jit
def kernel(user_id, movie_title, user_table, movie_table,
           W1, b1, W2, b2, W3, b3):
    uid = user_id.astype(jnp.int32)
    mid = movie_title.astype(jnp.int32)
    utab = user_table.T.reshape(-1)
    mtab = movie_table.T.reshape(-1)
    xu = _gather_u(uid, utab)
    xm0 = _gather_m0(mid, mtab)
    xm1 = _gather_m1(mid, mtab)
    args = (W1, b1, W2, b2, W3, b3)
    lo = _mlp_lo(xu, xm0, *args)
    hi = _mlp_hi(xu, xm1, *args)
    return jnp.concatenate([lo, hi], axis=1).T


# speedup vs baseline: 1.0854x; 1.0854x over previous
import jax
import jax.numpy as jnp
from jax import lax
from jax.experimental import pallas as pl
from jax.experimental.pallas import tpu as pltpu
from jax.experimental.pallas import tpu_sc as plsc

_B = 16384
_D = 32
_V = 100001
_NC = 2
_NS = 16
_NW = _NC * _NS
_BPW = _B // _NW
_CHUNK = 128
_NCHUNK = _BPW // _CHUNK


def _gather_body(idx_ref, tab_ref, xout_ref, idx_v, fidx_v, rows_v, sem):
    wid = lax.axis_index("s") * _NC + lax.axis_index("c")
    base = wid * _BPW
    pltpu.sync_copy(idx_ref.at[pl.ds(base, _BPW)], idx_v)
    for k in range(_BPW // 16):
        sl = pl.ds(16 * k, 16)
        v = idx_v[sl]
        for d in range(_D):
            fidx_v[d, sl] = v + jnp.int32(d * _V)
    copies = []
    for d in range(_D):
        copies.append(pltpu.async_copy(
            tab_ref.at[fidx_v.at[d]], rows_v.at[d], sem))
    for c in copies:
        c.wait()
    pltpu.sync_copy(rows_v, xout_ref.at[:, pl.ds(base, _BPW)])


def _make_gather():
    return pl.kernel(
        _gather_body,
        out_type=jax.ShapeDtypeStruct((_D, _B), jnp.float32),
        mesh=plsc.VectorSubcoreMesh(core_axis_name="c", subcore_axis_name="s"),
        scratch_types=[
            pltpu.VMEM((_BPW,), jnp.int32),
            pltpu.VMEM((_D, _BPW), jnp.int32),
            pltpu.VMEM((_D, _BPW), jnp.float32),
            pltpu.SemaphoreType.DMA,
        ],
        compiler_params=pltpu.CompilerParams(use_tc_tiling_on_sc=False),
    )


_gather_u = _make_gather()
_gather_m = _make_gather()


def _mlp_body(xu_ref, xm_ref, w1_ref, b1_ref, w2_ref, b2_ref,
              w3_ref, b3_ref, out_ref):
    cdims = (((0,), (0,)), ((), ()))
    h1 = jnp.maximum(
        lax.dot_general(w1_ref[:_D], xu_ref[...], cdims,
                        preferred_element_type=jnp.float32)
        + lax.dot_general(w1_ref[_D:], xm_ref[...], cdims,
                          preferred_element_type=jnp.float32)
        + b1_ref[...][:, None], 0.0)
    h2 = jnp.maximum(
        lax.dot_general(w2_ref[...], h1, cdims,
                        preferred_element_type=jnp.float32)
        + b2_ref[...][:, None], 0.0)
    out_ref[...] = (
        lax.dot_general(w3_ref[...], h2, cdims,
                        preferred_element_type=jnp.float32)
        + b3_ref[...][:, None])


_BB = 2048

_mlp = pl.pallas_call(
    _mlp_body,
    grid=(_B // _BB,),
    in_specs=[
        pl.BlockSpec((_D, _BB), lambda i: (0, i)),
        pl.BlockSpec((_D, _BB), lambda i: (0, i)),
        pl.BlockSpec((2 * _D, 256), lambda i: (0, 0)),
        pl.BlockSpec((256,), lambda i: (0,)),
        pl.BlockSpec((256, 64), lambda i: (0, 0)),
        pl.BlockSpec((64,), lambda i: (0,)),
        pl.BlockSpec((64, 1), lambda i: (0, 0)),
        pl.BlockSpec((1,), lambda i: (0,)),
    ],
    out_specs=pl.BlockSpec((1, _BB), lambda i: (0, i)),
    out_shape=jax.ShapeDtypeStruct((1, _B), jnp.float32),
)


@jax.jit
def kernel(user_id, movie_title, user_table, movie_table,
           W1, b1, W2, b2, W3, b3):
    xu = _gather_u(user_id.astype(jnp.int32), user_table.T.reshape(-1))
    xm = _gather_m(movie_title.astype(jnp.int32), movie_table.T.reshape(-1))
    out_t = _mlp(xu, xm, W1, b1, W2, b2, W3, b3)
    return out_t.T
